# Initial kernel scaffold; baseline (speedup 1.0000x reference)
import jax
import jax.numpy as jnp
from jax.experimental import pallas as pl


def kernel(x, node_table, edge_table, edge_index, constant):
    raise NotImplementedError("write your pallas kernel here")


# speedup vs baseline: 11.6004x; 11.6004x over previous
import functools

import jax
import jax.numpy as jnp
from jax import lax
from jax.experimental import pallas as pl
from jax.experimental.pallas import tpu as pltpu
from jax.experimental.pallas import tpu_sc as plsc

N = 10000
E = 160000
K = 16
B = 128

NC = 2
NS = 16
NW = NC * NS
L = 16

NB = B // L
NP = 10240
NODES_PER_W = NP // NW
EC = 128
NCHUNK = E // EC
FULL_ROUNDS = NCHUNK // NW
EXTRA = NCHUNK - FULL_ROUNDS * NW


def _body(xT_hbm, ntab_hbm, etab_hbm, src_hbm, dst_hbm, out_hbm,
          xn_v, nt_v, sidx_v, didx_v, xs_v, xd_v, et_v, accv, sem):
    cid = lax.axis_index("c")
    sid = lax.axis_index("s")
    wid = sid * NC + cid

    zero = jnp.zeros((L,), jnp.float32)
    accs0 = (zero,) * NB

    nbase = wid * NODES_PER_W
    pltpu.sync_copy(xT_hbm.at[pl.ds(nbase, NODES_PER_W)], xn_v)
    pltpu.sync_copy(ntab_hbm.at[pl.ds(nbase * K, NODES_PER_W * K)], nt_v)

    def nbody(i, accs):
        base = jnp.full((L,), i * K, jnp.int32)
        out = []
        for j in range(NB):
            s = xn_v[i, pl.ds(j * L, L)]
            v = plsc.load_gather(nt_v, [base + s])
            out.append(accs[j] + v)
        return tuple(out)

    accs = lax.fori_loop(0, NODES_PER_W, nbody, accs0)

    def echunk(g, accs, scale):
        lo = g * EC
        pltpu.sync_copy(src_hbm.at[pl.ds(lo, EC)], sidx_v)
        pltpu.sync_copy(dst_hbm.at[pl.ds(lo, EC)], didx_v)
        pltpu.async_copy(xT_hbm.at[sidx_v], xs_v, sem).wait()
        pltpu.async_copy(xT_hbm.at[didx_v], xd_v, sem).wait()
        pltpu.sync_copy(etab_hbm.at[pl.ds(lo * (K * K), EC * K * K)], et_v)

        def ebody(e, a):
            base = jnp.full((L,), e * (K * K), jnp.int32)
            out = []
            for j in range(NB):
                s = xs_v[e, pl.ds(j * L, L)]
                d = xd_v[e, pl.ds(j * L, L)]
                v = plsc.load_gather(et_v, [base + s * K + d])
                out.append(a[j] + v * scale)
            return tuple(out)

        return lax.fori_loop(0, EC, ebody, accs)

    one = jnp.ones((L,), jnp.float32)
    accs = lax.fori_loop(
        0, FULL_ROUNDS, lambda c, a: echunk(c * NW + wid, a, one), accs)

    g_extra = jnp.where(wid < EXTRA, FULL_ROUNDS * NW + wid, 0)
    scale = jnp.where(wid < EXTRA, one, jnp.zeros((L,), jnp.float32))
    accs = echunk(g_extra, accs, scale)

    for j in range(NB):
        accv[pl.ds(j * L, L)] = accs[j]
    pltpu.sync_copy(accv, out_hbm.at[wid])


@jax.jit
def _partials(xT, ntab_flat, etab_flat, src, dst):
    mesh = plsc.VectorSubcoreMesh(core_axis_name="c", subcore_axis_name="s",
                                  num_cores=NC, num_subcores=NS)
    run = pl.kernel(
        _body,
        out_type=jax.ShapeDtypeStruct((NW, B), jnp.float32),
        mesh=mesh,
        compiler_params=pltpu.CompilerParams(needs_layout_passes=False),
        scratch_types=[
            pltpu.VMEM((NODES_PER_W, B), jnp.int32),
            pltpu.VMEM((NODES_PER_W * K,), jnp.float32),
            pltpu.VMEM((EC,), jnp.int32),
            pltpu.VMEM((EC,), jnp.int32),
            pltpu.VMEM((EC, B), jnp.int32),
            pltpu.VMEM((EC, B), jnp.int32),
            pltpu.VMEM((EC * K * K,), jnp.float32),
            pltpu.VMEM((B,), jnp.float32),
            pltpu.SemaphoreType.DMA,
        ],
    )
    return run(xT, ntab_flat, etab_flat, src, dst)


def kernel(x, node_table, edge_table, edge_index, constant):
    xT = jnp.pad(x.T, ((0, NP - N), (0, 0)))
    ntab = jnp.pad(node_table, ((0, NP - N), (0, 0)))
    src = edge_index[0]
    dst = edge_index[1]
    parts = _partials(xT, ntab.reshape(-1), edge_table.reshape(-1),
                      src, dst)
    return parts.sum(axis=0) + constant
